# baseline (speedup 1.0000x reference)
import dataclasses
import functools

import jax
import jax.numpy as jnp
from jax import lax
from jax.experimental import pallas as pl
from jax.experimental.pallas import tpu as pltpu
from jax.experimental.pallas import tpu_sc as plsc

NE = 8192
D = 256
NT = 16384
BM = 1024
BN = 2048
NB_I = NT // BM
NB_J = NE // BN


def _prep_z_body(z_ref, zt_ref, z2_ref):
    z = z_ref[...]
    zt_ref[...] = (z.astype(jnp.bfloat16) * 2).T
    z2_ref[...] = jnp.sum(z * z)[None, None, None]


def _prep_w_body(w_ref, wb_ref, w2_ref):
    w = w_ref[...]
    wb_ref[...] = w.astype(jnp.bfloat16)
    w2_ref[...] = jnp.sum(w * w, axis=1, keepdims=True)


_SPS = 32


def _vq_body(zt_ref, wb_ref, w2_ref, idx_ref, part_ref, rmin_ref, rarg_ref,
             zw_scr):
    j = pl.program_id(1)
    zw_scr[...] = lax.dot_general(
        wb_ref[...], zt_ref[...], (((1,), (0,)), ((), ())),
        preferred_element_type=jnp.float32)

    def _stripe_scan(s, carry):
        acc_min, acc_s = carry
        for t in range(_SPS // 8):
            r0 = s * _SPS + t * 8
            blk = w2_ref[pl.ds(r0, 8), :] - zw_scr[pl.ds(r0, 8), :]
            m = blk < acc_min
            acc_s = jnp.where(m, jnp.full((8, BM), r0, jnp.int32), acc_s)
            acc_min = jnp.minimum(acc_min, blk)
        return acc_min, acc_s

    acc_min, acc_s = lax.fori_loop(
        0, BN // _SPS, _stripe_scan,
        (jnp.full((8, BM), jnp.inf, jnp.float32),
         jnp.zeros((8, BM), jnp.int32)),
        unroll=False)

    lmin = jnp.min(acc_min, axis=0, keepdims=True)
    iota8 = lax.broadcasted_iota(jnp.int32, (8, BM), 0)
    larg = jnp.min(jnp.where(acc_min == lmin, acc_s + iota8, BN),
                   axis=0, keepdims=True) + j * BN

    @pl.when(j == 0)
    def _():
        rmin_ref[...] = lmin
        rarg_ref[...] = larg

    @pl.when(j > 0)
    def _():
        better = lmin < rmin_ref[...]
        rarg_ref[...] = jnp.where(better, larg, rarg_ref[...])
        rmin_ref[...] = jnp.where(better, lmin, rmin_ref[...])

    @pl.when(j == NB_J - 1)
    def _():
        idx_ref[...] = rarg_ref[...].reshape(1, 1, BM)
        part_ref[...] = jnp.sum(rmin_ref[...])[None, None, None]


_prep_z = pl.pallas_call(
    _prep_z_body,
    grid=(NB_I,),
    in_specs=[pl.BlockSpec((BM, D), lambda i: (i, 0))],
    out_specs=[
        pl.BlockSpec((D, BM), lambda i: (0, i)),
        pl.BlockSpec((1, 1, 1), lambda i: (i, 0, 0)),
    ],
    out_shape=[
        jax.ShapeDtypeStruct((D, NT), jnp.bfloat16),
        jax.ShapeDtypeStruct((NB_I, 1, 1), jnp.float32),
    ],
)

_prep_w = pl.pallas_call(
    _prep_w_body,
    grid=(NB_J,),
    in_specs=[pl.BlockSpec((BN, D), lambda j: (j, 0))],
    out_specs=[
        pl.BlockSpec((BN, D), lambda j: (j, 0)),
        pl.BlockSpec((BN, 1), lambda j: (j, 0)),
    ],
    out_shape=[
        jax.ShapeDtypeStruct((NE, D), jnp.bfloat16),
        jax.ShapeDtypeStruct((NE, 1), jnp.float32),
    ],
)

_vq = pl.pallas_call(
    _vq_body,
    grid=(NB_I, NB_J),
    in_specs=[
        pl.BlockSpec((D, BM), lambda i, j: (0, i)),
        pl.BlockSpec((BN, D), lambda i, j: (j, 0)),
        pl.BlockSpec((BN, 1), lambda i, j: (j, 0)),
    ],
    out_specs=[
        pl.BlockSpec((1, 1, BM), lambda i, j: (i, 0, 0)),
        pl.BlockSpec((1, 1, 1), lambda i, j: (i, 0, 0)),
    ],
    out_shape=[
        jax.ShapeDtypeStruct((NB_I, 1, BM), jnp.int32),
        jax.ShapeDtypeStruct((NB_I, 1, 1), jnp.float32),
    ],
    scratch_shapes=[
        pltpu.VMEM((1, BM), jnp.float32),
        pltpu.VMEM((1, BM), jnp.int32),
        pltpu.VMEM((BN, BM), jnp.float32),
    ],
    compiler_params=pltpu.CompilerParams(
        dimension_semantics=("parallel", "arbitrary")),
)


_NW = 32
_RPW = NT // _NW
_CH = 128
_SAMPLES = 16

def _sc_gather_div_body(w_hbm, idx_hbm, zq_hbm, cnt_hbm,
                        idx_v, rows_a, rows_b, sidx_v, flags_v, out16_v,
                        sem_a, sem_b):
    wid = lax.axis_index("s") * 2 + lax.axis_index("c")
    base = wid * _RPW
    pltpu.sync_copy(idx_hbm.at[pl.ds(base, _RPW)], idx_v)
    cp_a = pltpu.async_copy(w_hbm.at[idx_v.at[pl.ds(0, _CH)]], rows_a, sem_a)
    cp_b = pltpu.async_copy(w_hbm.at[idx_v.at[pl.ds(_CH, _CH)]], rows_b, sem_b)
    cp_a.wait()
    pltpu.sync_copy(rows_a, zq_hbm.at[pl.ds(base, _CH)])
    cp_a = pltpu.async_copy(w_hbm.at[idx_v.at[pl.ds(2 * _CH, _CH)]], rows_a,
                            sem_a)
    cp_b.wait()
    pltpu.sync_copy(rows_b, zq_hbm.at[pl.ds(base + _CH, _CH)])
    cp_b = pltpu.async_copy(w_hbm.at[idx_v.at[pl.ds(3 * _CH, _CH)]], rows_b,
                            sem_b)
    cp_a.wait()
    pltpu.sync_copy(rows_a, zq_hbm.at[pl.ds(base + 2 * _CH, _CH)])
    cp_b.wait()
    pltpu.sync_copy(rows_b, zq_hbm.at[pl.ds(base + 3 * _CH, _CH)])

    @pl.when(wid < _SAMPLES)
    def _():
        pltpu.sync_copy(idx_hbm.at[pl.ds(wid * BM, BM)], sidx_v)

        @pl.loop(0, NE, step=16)
        def _(k):
            flags_v[pl.ds(k, 16)] = jnp.zeros((16,), jnp.float32)

        ones = jnp.ones((16,), jnp.float32)

        @pl.loop(0, BM, step=16)
        def _(t):
            plsc.store_scatter(flags_v, [sidx_v[pl.ds(t, 16)]], ones)

        def _body(k, acc):
            return acc + flags_v[pl.ds(k * 16, 16)]

        acc = lax.fori_loop(0, NE // 16, _body, jnp.zeros((16,), jnp.float32))
        out16_v[...] = jnp.full((16,), jnp.sum(acc), jnp.float32)
        pltpu.sync_copy(out16_v, cnt_hbm.at[wid])


@functools.cache
def _sc_gather_div_call():
    params = pltpu.CompilerParams()
    if "needs_layout_passes" in pltpu.CompilerParams.__dataclass_fields__:
        params = dataclasses.replace(params, needs_layout_passes=False)
    return pl.kernel(
        _sc_gather_div_body,
        mesh=plsc.VectorSubcoreMesh(core_axis_name="c", subcore_axis_name="s"),
        compiler_params=params,
        out_type=[
            jax.ShapeDtypeStruct((NT, D), jnp.float32),
            jax.ShapeDtypeStruct((_SAMPLES, 16), jnp.float32),
        ],
        scratch_types=[
            pltpu.VMEM((_RPW,), jnp.int32),
            pltpu.VMEM((_CH, D), jnp.float32),
            pltpu.VMEM((_CH, D), jnp.float32),
            pltpu.VMEM((BM,), jnp.int32),
            pltpu.VMEM((NE,), jnp.float32),
            pltpu.VMEM((16,), jnp.float32),
            pltpu.SemaphoreType.DMA,
            pltpu.SemaphoreType.DMA,
        ],
    )


def kernel(z, weight):
    zf = z.reshape(NT, D)
    zt, z2p = _prep_z(zf)
    wb, w2 = _prep_w(weight)
    idx3, part = _vq(zt, wb, w2)
    idxf = idx3.reshape(NT)
    index = idx3.reshape(16, 32, 32)

    z_q, cnt = _sc_gather_div_call()(weight, idxf)
    z_q_st = z_q.reshape(16, 32, 32, D)
    diversity = jnp.sum(cnt[:, 0]) / float(NT)

    loss = (jnp.sum(z2p) + jnp.sum(part)) * (1.25 / float(NT * D))
    return (z_q_st, index, loss, diversity)


# speedup vs baseline: 3.9474x; 3.9474x over previous
import dataclasses
import functools

import jax
import jax.numpy as jnp
from jax import lax
from jax.experimental import pallas as pl
from jax.experimental.pallas import tpu as pltpu
from jax.experimental.pallas import tpu_sc as plsc

NE = 8192
D = 256
NT = 16384
BM = 1024
BN = 4096
NB_I = NT // BM
NB_J = NE // BN


_SPS = 32


def _vq_body(z_ref, w_ref, idx_ref, part_ref,
             rmin_ref, rarg_ref, zw_scr, zt_scr, wb_scr, w2_scr):
    i = pl.program_id(0)
    j = pl.program_id(1)

    @pl.when(j == 0)
    def _():
        zt_scr[...] = (z_ref[...].astype(jnp.bfloat16) * 2).T

    @pl.when(i == 0)
    def _():
        w = w_ref[...]
        wb_scr[pl.ds(j * BN, BN), :] = w.astype(jnp.bfloat16)
        w2_scr[pl.ds(j * BN, BN), :] = jnp.sum(w * w, axis=1, keepdims=True)

    zw_scr[...] = lax.dot_general(
        wb_scr[pl.ds(j * BN, BN), :], zt_scr[...], (((1,), (0,)), ((), ())),
        preferred_element_type=jnp.float32)

    def _stripe_scan(s, carry):
        acc_min, acc_s = carry
        for t in range(_SPS // 8):
            r0 = s * _SPS + t * 8
            blk = (w2_scr[pl.ds(j * BN + r0, 8), :]
                   - zw_scr[pl.ds(r0, 8), :])
            m = blk < acc_min
            acc_s = jnp.where(m, jnp.full((8, BM), r0, jnp.int32), acc_s)
            acc_min = jnp.minimum(acc_min, blk)
        return acc_min, acc_s

    acc_min, acc_s = lax.fori_loop(
        0, BN // _SPS, _stripe_scan,
        (jnp.full((8, BM), jnp.inf, jnp.float32),
         jnp.zeros((8, BM), jnp.int32)),
        unroll=True)

    lmin = jnp.min(acc_min, axis=0, keepdims=True)
    iota8 = lax.broadcasted_iota(jnp.int32, (8, BM), 0)
    larg = jnp.min(jnp.where(acc_min == lmin, acc_s + iota8, BN),
                   axis=0, keepdims=True) + j * BN

    @pl.when(j == 0)
    def _():
        rmin_ref[...] = lmin
        rarg_ref[...] = larg

    @pl.when(j > 0)
    def _():
        better = lmin < rmin_ref[...]
        rarg_ref[...] = jnp.where(better, larg, rarg_ref[...])
        rmin_ref[...] = jnp.where(better, lmin, rmin_ref[...])

    @pl.when(j == NB_J - 1)
    def _():
        idx_ref[...] = rarg_ref[...].reshape(1, 1, BM)
        z = z_ref[...]
        part_ref[...] = (jnp.sum(z * z)
                         + jnp.sum(rmin_ref[...]))[None, None, None]


_vq = pl.pallas_call(
    _vq_body,
    grid=(NB_I, NB_J),
    in_specs=[
        pl.BlockSpec((BM, D), lambda i, j: (i, 0)),
        pl.BlockSpec((BN, D), lambda i, j: (j * (1 - jnp.minimum(i, 1)), 0)),
    ],
    out_specs=[
        pl.BlockSpec((1, 1, BM), lambda i, j: (i, 0, 0)),
        pl.BlockSpec((1, 1, 1), lambda i, j: (i, 0, 0)),
    ],
    out_shape=[
        jax.ShapeDtypeStruct((NB_I, 1, BM), jnp.int32),
        jax.ShapeDtypeStruct((NB_I, 1, 1), jnp.float32),
    ],
    scratch_shapes=[
        pltpu.VMEM((1, BM), jnp.float32),
        pltpu.VMEM((1, BM), jnp.int32),
        pltpu.VMEM((BN, BM), jnp.float32),
        pltpu.VMEM((D, BM), jnp.bfloat16),
        pltpu.VMEM((NE, D), jnp.bfloat16),
        pltpu.VMEM((NE, 1), jnp.float32),
    ],
)


_NW = 32
_CH = 128
_SAMPLES = 16
_SAMP = NT // _SAMPLES
_CH_DIV = 2
_CH_GAT = 6
_DIV_ROWS = _CH_DIV * _CH * _SAMPLES


def _gather_chunks(w_hbm, idx_hbm, zq_hbm, idx_v, rows_a, rows_b,
                   sem_a, sem_b, base, nch):
    pltpu.sync_copy(idx_hbm.at[pl.ds(base, nch * _CH)], idx_v.at[pl.ds(0, nch * _CH)])
    bufs = (rows_a, rows_b)
    sems = (sem_a, sem_b)
    cps = [None, None]
    cps[0] = pltpu.async_copy(w_hbm.at[idx_v.at[pl.ds(0, _CH)]], rows_a, sem_a)
    if nch > 1:
        cps[1] = pltpu.async_copy(w_hbm.at[idx_v.at[pl.ds(_CH, _CH)]], rows_b,
                                  sem_b)
    for c in range(nch):
        p = c % 2
        cps[p].wait()
        pltpu.sync_copy(bufs[p], zq_hbm.at[pl.ds(base + c * _CH, _CH)])
        if c + 2 < nch:
            cps[p] = pltpu.async_copy(
                w_hbm.at[idx_v.at[pl.ds((c + 2) * _CH, _CH)]], bufs[p],
                sems[p])


def _sc_gather_div_body(w_hbm, idx_hbm, zq_hbm, cnt_hbm,
                        idx_v, rows_a, rows_b, sidx_v, flags_v, out16_v,
                        sem_a, sem_b):
    wid = lax.axis_index("s") * 2 + lax.axis_index("c")

    @pl.when(wid >= _SAMPLES)
    def _():
        base = _DIV_ROWS + (wid - _SAMPLES) * (_CH_GAT * _CH)
        _gather_chunks(w_hbm, idx_hbm, zq_hbm, idx_v, rows_a, rows_b,
                       sem_a, sem_b, base, _CH_GAT)

    @pl.when(wid < _SAMPLES)
    def _():
        _gather_chunks(w_hbm, idx_hbm, zq_hbm, idx_v, rows_a, rows_b,
                       sem_a, sem_b, wid * (_CH_DIV * _CH), _CH_DIV)
        pltpu.sync_copy(idx_hbm.at[pl.ds(wid * _SAMP, _SAMP)], sidx_v)

        @pl.loop(0, NE, step=16)
        def _(k):
            flags_v[pl.ds(k, 16)] = jnp.zeros((16,), jnp.float32)

        ones = jnp.ones((16,), jnp.float32)

        @pl.loop(0, _SAMP, step=16)
        def _(t):
            plsc.store_scatter(flags_v, [sidx_v[pl.ds(t, 16)]], ones)

        def _body(k, acc):
            return acc + flags_v[pl.ds(k * 16, 16)]

        acc = lax.fori_loop(0, NE // 16, _body, jnp.zeros((16,), jnp.float32))
        out16_v[...] = jnp.full((16,), jnp.sum(acc), jnp.float32)
        pltpu.sync_copy(out16_v, cnt_hbm.at[wid])


@functools.cache
def _sc_gather_div_call():
    params = pltpu.CompilerParams()
    if "needs_layout_passes" in pltpu.CompilerParams.__dataclass_fields__:
        params = dataclasses.replace(params, needs_layout_passes=False)
    return pl.kernel(
        _sc_gather_div_body,
        mesh=plsc.VectorSubcoreMesh(core_axis_name="c", subcore_axis_name="s"),
        compiler_params=params,
        out_type=[
            jax.ShapeDtypeStruct((NT, D), jnp.float32),
            jax.ShapeDtypeStruct((_SAMPLES, 16), jnp.float32),
        ],
        scratch_types=[
            pltpu.VMEM((_CH_GAT * _CH,), jnp.int32),
            pltpu.VMEM((_CH, D), jnp.float32),
            pltpu.VMEM((_CH, D), jnp.float32),
            pltpu.VMEM((_SAMP,), jnp.int32),
            pltpu.VMEM((NE,), jnp.float32),
            pltpu.VMEM((16,), jnp.float32),
            pltpu.SemaphoreType.DMA,
            pltpu.SemaphoreType.DMA,
        ],
    )


def kernel(z, weight):
    zf = z.reshape(NT, D)
    idx3, part = _vq(zf, weight)
    idxf = idx3.reshape(NT)
    index = idx3.reshape(16, 32, 32)

    z_q, cnt = _sc_gather_div_call()(weight, idxf)
    z_q_st = z_q.reshape(16, 32, 32, D)
    diversity = jnp.sum(cnt[:, 0]) / float(NT)

    loss = jnp.sum(part) * (1.25 / float(NT * D))
    return (z_q_st, index, loss, diversity)
